# baseline (speedup 1.0000x reference)
import jax
import jax.numpy as jnp
from jax.experimental import pallas as pl
from jax.experimental.pallas import tpu as pltpu

B, N, D, H, T = 8, 1024, 128, 32, 3
NSHARDS = 4
RS = N // NSHARDS


def _dot(a, b):
    return jax.lax.dot_general(
        a, b,
        (((a.ndim - 1,), (0,)), ((), ())),
        preferred_element_type=jnp.float32)


def _gru(a, x, wihT, bih, whhT, bhh):
    gi = _dot(a, wihT) + bih
    gh = _dot(x, whhT) + bhh
    r = jax.nn.sigmoid(gi[:, :H] + gh[:, :H])
    z = jax.nn.sigmoid(gi[:, H:2 * H] + gh[:, H:2 * H])
    n = jnp.tanh(gi[:, 2 * H:] + r * gh[:, 2 * H:])
    return (1.0 - z) * n + z * x


def _body(x_padded_ref, e0_ref, e1_ref, e2_ref, e3_ref, fc_wT_ref, fc_b_ref,
          W1c_ref, wih1T_ref, whh1T_ref, bih1_ref, bhh1_ref,
          W2c_ref, wih2T_ref, whh2T_ref, bih2_ref, bhh2_ref,
          W3c_ref, wih3T_ref, whh3T_ref, bih3_ref, bhh3_ref,
          out_wT_ref, out_b_ref, out_ref, x_s, m_s):
    e_refs = (e0_ref, e1_ref, e2_ref, e3_ref)
    x_s[...] = _dot(x_padded_ref[0], fc_wT_ref[:]) + fc_b_ref[:]

    for (Wc_ref, wihT_ref, whhT_ref, bih_ref, bhh_ref) in (
            (W1c_ref, wih1T_ref, whh1T_ref, bih1_ref, bhh1_ref),
            (W2c_ref, wih2T_ref, whh2T_ref, bih2_ref, bhh2_ref)):
        m_s[...] = _dot(x_s[...], Wc_ref[:])
        for i, e_ref in enumerate(e_refs):
            acc = _dot(e_ref[0, 0], m_s[:, :H])
            acc += _dot(e_ref[0, 1], m_s[:, H:2 * H])
            acc += _dot(e_ref[0, 2], m_s[:, 2 * H:])
            rows = slice(i * RS, (i + 1) * RS)
            x_s[rows, :] = _gru(acc, x_s[rows, :], wihT_ref[:], bih_ref[:],
                                whhT_ref[:], bhh_ref[:])

    m_s[...] = _dot(x_s[...], W3c_ref[:])
    a3 = _dot(e0_ref[0, 0, 5:6, :], m_s[:, :H])
    a3 += _dot(e0_ref[0, 1, 5:6, :], m_s[:, H:2 * H])
    a3 += _dot(e0_ref[0, 2, 5:6, :], m_s[:, 2 * H:])
    h = _gru(a3, x_s[5:6, :], wih3T_ref[:], bih3_ref[:],
             whh3T_ref[:], bhh3_ref[:])

    logits = _dot(h, out_wT_ref[:]) + out_b_ref[:]
    mx = jnp.max(logits, axis=1, keepdims=True)
    lse = mx + jnp.log(jnp.sum(jnp.exp(logits - mx), axis=1, keepdims=True))
    out_ref[0] = logits - lse


def _shard_spec(i):
    return pl.BlockSpec((1, T, RS, N), lambda b, i=i: (b, 0, i, 0))


@jax.jit
def kernel(x_padded, x_lengths, edges, fc_w, fc_b,
           W1, wih1, whh1, bih1, bhh1,
           W2, wih2, whh2, bih2, bhh2,
           W3, wih3, whh3, bih3, bhh3,
           out_w, out_b):
    del x_lengths

    def full(x):
        return pl.BlockSpec(x.shape, lambda b: (0,) * x.ndim)

    row2 = lambda v: v.reshape(1, -1)
    wcat = lambda W: W.transpose(1, 0, 2).reshape(H, T * H)
    ins = (
        x_padded, edges, edges, edges, edges,
        fc_w.T, row2(fc_b),
        wcat(W1), wih1.T, whh1.T, row2(bih1), row2(bhh1),
        wcat(W2), wih2.T, whh2.T, row2(bih2), row2(bhh2),
        wcat(W3), wih3.T, whh3.T, row2(bih3), row2(bhh3),
        out_w.T, row2(out_b),
    )
    specs = [
        pl.BlockSpec((1, N, D), lambda b: (b, 0, 0)),
    ] + [_shard_spec(i) for i in range(NSHARDS)] + [full(x) for x in ins[5:]]

    out = pl.pallas_call(
        _body,
        grid=(B,),
        in_specs=specs,
        out_specs=pl.BlockSpec((1, 1, 5), lambda b: (b, 0, 0)),
        out_shape=jax.ShapeDtypeStruct((B, 1, 5), jnp.float32),
        scratch_shapes=[pltpu.VMEM((N, H), jnp.float32),
                        pltpu.VMEM((N, T * H), jnp.float32)],
        compiler_params=pltpu.CompilerParams(
            dimension_semantics=("arbitrary",)),
    )(*ins)
    return out.reshape(B, 5)


# speedup vs baseline: 1.3805x; 1.3805x over previous
import jax
import jax.numpy as jnp
from jax.experimental import pallas as pl
from jax.experimental.pallas import tpu as pltpu

B, N, D, H, T = 8, 1024, 128, 32, 3
NSHARDS = 4
RS = N // NSHARDS


def _dot(a, b):
    return jax.lax.dot_general(
        a, b,
        (((a.ndim - 1,), (0,)), ((), ())),
        preferred_element_type=jnp.float32)


def _gru(a, x, wihT, bih, whhT, bhh):
    gi = _dot(a, wihT) + bih
    gh = _dot(x, whhT) + bhh
    r = jax.nn.sigmoid(gi[:, :H] + gh[:, :H])
    z = jax.nn.sigmoid(gi[:, H:2 * H] + gh[:, H:2 * H])
    n = jnp.tanh(gi[:, 2 * H:] + r * gh[:, 2 * H:])
    return (1.0 - z) * n + z * x


def _body(x_padded_ref, e0_ref, e1_ref, e2_ref, e3_ref, fc_wT_ref, fc_b_ref,
          W1c_ref, wih1T_ref, whh1T_ref, bih1_ref, bhh1_ref,
          W2c_ref, wih2T_ref, whh2T_ref, bih2_ref, bhh2_ref,
          W3c_ref, wih3T_ref, whh3T_ref, bih3_ref, bhh3_ref,
          out_wT_ref, out_b_ref, out_ref, x_s, a_s):
    e_refs = (e0_ref, e1_ref, e2_ref, e3_ref)
    x_s[...] = _dot(x_padded_ref[0], fc_wT_ref[:]) + fc_b_ref[:]

    for (Wc_ref, wihT_ref, whhT_ref, bih_ref, bhh_ref) in (
            (W1c_ref, wih1T_ref, whh1T_ref, bih1_ref, bhh1_ref),
            (W2c_ref, wih2T_ref, whh2T_ref, bih2_ref, bhh2_ref)):
        mall = _dot(x_s[...], Wc_ref[:])
        m0, m1, m2 = mall[:, :H], mall[:, H:2 * H], mall[:, 2 * H:]
        for i, e_ref in enumerate(e_refs):
            ai = _dot(e_ref[0, 0], m0)
            ai += _dot(e_ref[0, 1], m1)
            ai += _dot(e_ref[0, 2], m2)
            a_s[i * RS:(i + 1) * RS, :] = ai
        x_s[...] = _gru(a_s[...], x_s[...], wihT_ref[:], bih_ref[:],
                        whhT_ref[:], bhh_ref[:])

    mall = _dot(x_s[...], W3c_ref[:])
    a3 = _dot(e0_ref[0, 0, 5:6, :], mall[:, :H])
    a3 += _dot(e0_ref[0, 1, 5:6, :], mall[:, H:2 * H])
    a3 += _dot(e0_ref[0, 2, 5:6, :], mall[:, 2 * H:])
    h = _gru(a3, x_s[5:6, :], wih3T_ref[:], bih3_ref[:],
             whh3T_ref[:], bhh3_ref[:])

    logits = _dot(h, out_wT_ref[:]) + out_b_ref[:]
    mx = jnp.max(logits, axis=1, keepdims=True)
    lse = mx + jnp.log(jnp.sum(jnp.exp(logits - mx), axis=1, keepdims=True))
    out_ref[0] = logits - lse


def _shard_spec(i):
    return pl.BlockSpec((1, T, RS, N), lambda b, i=i: (b, 0, i, 0))


@jax.jit
def kernel(x_padded, x_lengths, edges, fc_w, fc_b,
           W1, wih1, whh1, bih1, bhh1,
           W2, wih2, whh2, bih2, bhh2,
           W3, wih3, whh3, bih3, bhh3,
           out_w, out_b):
    del x_lengths

    def full(x):
        return pl.BlockSpec(x.shape, lambda b: (0,) * x.ndim)

    row2 = lambda v: v.reshape(1, -1)
    wcat = lambda W: W.transpose(1, 0, 2).reshape(H, T * H)
    ins = (
        x_padded, edges, edges, edges, edges,
        fc_w.T, row2(fc_b),
        wcat(W1), wih1.T, whh1.T, row2(bih1), row2(bhh1),
        wcat(W2), wih2.T, whh2.T, row2(bih2), row2(bhh2),
        wcat(W3), wih3.T, whh3.T, row2(bih3), row2(bhh3),
        out_w.T, row2(out_b),
    )
    specs = [
        pl.BlockSpec((1, N, D), lambda b: (b, 0, 0)),
    ] + [_shard_spec(i) for i in range(NSHARDS)] + [full(x) for x in ins[5:]]

    out = pl.pallas_call(
        _body,
        grid=(B,),
        in_specs=specs,
        out_specs=pl.BlockSpec((1, 1, 5), lambda b: (b, 0, 0)),
        out_shape=jax.ShapeDtypeStruct((B, 1, 5), jnp.float32),
        scratch_shapes=[pltpu.VMEM((N, H), jnp.float32),
                        pltpu.VMEM((N, H), jnp.float32)],
        compiler_params=pltpu.CompilerParams(
            dimension_semantics=("arbitrary",)),
    )(*ins)
    return out.reshape(B, 5)
